# baseline (speedup 1.0000x reference)
import jax
import jax.numpy as jnp
from jax import lax
from jax.experimental import pallas as pl
from jax.experimental.pallas import tpu as pltpu
from jax.experimental.pallas import tpu_sc as plsc

NUM_RELATIONS = 1000
DIM = 128
BATCH = 4096
L = 16
NW = 32
S = 4
GPT_MAX = 56
MIDROWS = GPT_MAX + 2
SLOTS = GPT_MAX * S
HALF = SLOTS // 2
NBLK = DIM // L


def _compute_group(ht_v, mat_v, out_v, s0):
    t_vecs = [
        [ht_v[s0 + i, pl.ds(DIM + L * j, L)] for j in range(NBLK)]
        for i in range(S)
    ]

    def blk_body(db, accs):
        hv = [ht_v[s0 + i, pl.ds(db * L, L)] for i in range(S)]
        accs = list(accs)
        for k in range(L):
            row = db * L + k
            m = [mat_v[0, row, pl.ds(L * j, L)] for j in range(NBLK)]
            for i in range(S):
                dot = m[0] * t_vecs[i][0]
                for j in range(1, NBLK):
                    dot = dot + m[j] * t_vecs[i][j]
                accs[i] = accs[i] + hv[i][k] * dot
        return tuple(accs)

    zero = jnp.zeros((L,), jnp.float32)
    accs = lax.fori_loop(0, NBLK, blk_body, (zero,) * S)
    for i in range(S):
        out_v[s0 + i] = accs[i]


def _sc_body(ht_hbm, iid_hbm, mid_hbm, cnt_hbm, table_hbm, out_hbm,
             iid_v, mid_v, cnt_v, ht_v, mat0_v, mat1_v, out_v,
             sem_h, sem0, sem1):
    cid = lax.axis_index("c")
    sid = lax.axis_index("s")
    wid = sid * 2 + cid

    pltpu.sync_copy(iid_hbm.at[wid], iid_v)
    pltpu.sync_copy(mid_hbm.at[wid], mid_v)
    pltpu.sync_copy(cnt_hbm, cnt_v)
    gpt = cnt_v[pl.ds(0, L)][0]

    pltpu.async_copy(ht_hbm.at[iid_v.at[0]], ht_v.at[pl.ds(0, HALF)], sem_h)
    pltpu.async_copy(ht_hbm.at[iid_v.at[1]], ht_v.at[pl.ds(HALF, HALF)], sem_h)

    pltpu.async_copy(table_hbm.at[mid_v.at[0]], mat0_v, sem0)
    pltpu.async_copy(table_hbm.at[mid_v.at[1]], mat1_v, sem1)

    pltpu.make_async_copy(ht_hbm.at[iid_v.at[0]], ht_v.at[pl.ds(0, HALF)], sem_h).wait()
    pltpu.make_async_copy(ht_hbm.at[iid_v.at[1]], ht_v.at[pl.ds(HALF, HALF)], sem_h).wait()

    def pair_body(p, _):
        g0 = 2 * p
        pltpu.make_async_copy(table_hbm.at[mid_v.at[g0]], mat0_v, sem0).wait()
        _compute_group(ht_v, mat0_v, out_v, g0 * S)
        pltpu.async_copy(table_hbm.at[mid_v.at[g0 + 2]], mat0_v, sem0)

        pltpu.make_async_copy(table_hbm.at[mid_v.at[g0 + 1]], mat1_v, sem1).wait()
        _compute_group(ht_v, mat1_v, out_v, (g0 + 1) * S)
        pltpu.async_copy(table_hbm.at[mid_v.at[g0 + 3]], mat1_v, sem1)
        return 0

    lax.fori_loop(0, gpt // 2, pair_body, 0)

    pltpu.make_async_copy(table_hbm.at[mid_v.at[gpt]], mat0_v, sem0).wait()
    pltpu.make_async_copy(table_hbm.at[mid_v.at[gpt + 1]], mat1_v, sem1).wait()

    pltpu.sync_copy(out_v, out_hbm.at[wid])


@jax.jit
def _bilinear_sc(ht, iid, mid, cnt, table):
    mesh = plsc.VectorSubcoreMesh(core_axis_name="c", subcore_axis_name="s")
    fn = pl.kernel(
        _sc_body,
        out_type=jax.ShapeDtypeStruct((NW, SLOTS, L), jnp.float32),
        mesh=mesh,
        scratch_types=[
            pltpu.VMEM((2, HALF), jnp.int32),
            pltpu.VMEM((MIDROWS, 1), jnp.int32),
            pltpu.VMEM((L,), jnp.int32),
            pltpu.VMEM((SLOTS, 2 * DIM), jnp.float32),
            pltpu.VMEM((1, DIM, DIM), jnp.float32),
            pltpu.VMEM((1, DIM, DIM), jnp.float32),
            pltpu.VMEM((SLOTS, L), jnp.float32),
            pltpu.SemaphoreType.DMA,
            pltpu.SemaphoreType.DMA,
            pltpu.SemaphoreType.DMA,
        ],
    )
    return fn(ht, iid, mid, cnt, table)


def kernel(heads_and_tails, relations, kernel):
    rel = relations[:, 0].astype(jnp.int32)

    order = jnp.argsort(rel)
    srel = rel[order]
    counts = jnp.bincount(rel, length=NUM_RELATIONS)
    ng = (counts + (S - 1)) // S
    gbase = jnp.cumsum(ng) - ng
    total_g = jnp.sum(ng)
    gpt = ((total_g + 2 * NW - 1) // (2 * NW)) * 2
    segstart = jnp.cumsum(counts) - counts
    rank = jnp.arange(BATCH, dtype=jnp.int32) - segstart[srel]
    g_global = gbase[srel] + rank // S
    tile = g_global // gpt
    g_local = g_global % gpt
    flat = tile * SLOTS + g_local * S + rank % S

    iid = jnp.zeros((NW * SLOTS,), jnp.int32).at[flat].set(order.astype(jnp.int32))
    mid = jnp.zeros((NW * MIDROWS,), jnp.int32).at[tile * MIDROWS + g_local].set(srel)
    outpos = jnp.zeros((BATCH,), jnp.int32).at[order].set(flat)
    cnt = jnp.full((L,), gpt, jnp.int32)

    out16 = _bilinear_sc(
        heads_and_tails,
        iid.reshape(NW, 2, HALF),
        mid.reshape(NW, MIDROWS, 1),
        cnt,
        kernel,
    )
    out = jnp.sum(out16.reshape(NW * SLOTS, L), axis=1)[outpos]
    return out[:, None]
